# baseline (speedup 1.0000x reference)
import functools

import jax
import jax.numpy as jnp
from jax import lax
from jax.experimental import pallas as pl
from jax.experimental.pallas import tpu as pltpu
from jax.experimental.pallas import tpu_sc as plsc

_NC = 2
_NS = 16
_NW = _NC * _NS
_LANES = 16


def _build(N, T, D):
    rows_per_w = N // _NW
    seqs_per_w = rows_per_w // T
    SEQ_PER_CHUNK = 2
    CH_ROWS = SEQ_PER_CHUNK * T
    n_chunks = seqs_per_w // SEQ_PER_CHUNK
    G = 80
    n_g = CH_ROWS // G

    mesh = plsc.VectorSubcoreMesh(
        core_axis_name="c", subcore_axis_name="s",
        num_cores=_NC, num_subcores=_NS,
    )

    @functools.partial(
        pl.kernel,
        out_type=jax.ShapeDtypeStruct((N, D), jnp.float32),
        mesh=mesh,
        scratch_types=[
            pltpu.VMEM((T, D), jnp.float32),
            pltpu.VMEM((CH_ROWS,), jnp.int32),
            pltpu.VMEM((CH_ROWS, D), jnp.float32),
            pltpu.SemaphoreType.DMA,
        ],
        compiler_params=pltpu.CompilerParams(use_tc_tiling_on_sc=False),
    )
    def emb_kernel(ids_hbm, tok_hbm, pos_hbm, out_hbm, pos_v, idx_v, rows_v, sem):
        wid = lax.axis_index("s") * _NC + lax.axis_index("c")
        base = pl.multiple_of(wid * rows_per_w, CH_ROWS)
        pltpu.sync_copy(pos_hbm, pos_v)

        def chunk_body(ci, carry):
            row0 = pl.multiple_of(base + ci * CH_ROWS, CH_ROWS)
            pltpu.sync_copy(ids_hbm.at[pl.ds(row0, CH_ROWS)], idx_v)
            copies = []
            for g in range(n_g):
                c = pltpu.make_async_copy(
                    tok_hbm.at[idx_v.at[pl.ds(g * G, G)]],
                    rows_v.at[pl.ds(g * G, G)],
                    sem,
                )
                c.start()
                copies.append(c)
            for c in copies:
                c.wait()

            def add_body(t, carry2):
                for cc in range(D // _LANES):
                    v = pos_v[t, pl.ds(cc * _LANES, _LANES)]
                    for s in range(SEQ_PER_CHUNK):
                        plsc.addupdate(
                            rows_v.at[s * T + t, pl.ds(cc * _LANES, _LANES)], v
                        )
                return carry2

            lax.fori_loop(0, T, add_body, 0)
            pltpu.sync_copy(rows_v, out_hbm.at[pl.ds(row0, CH_ROWS)])
            return carry

        lax.fori_loop(0, n_chunks, chunk_body, 0)

    return emb_kernel


def kernel(token_ids, tok_table, pos_table):
    B, T = token_ids.shape
    V, D = tok_table.shape
    N = B * T
    ids_flat = token_ids.reshape(N).astype(jnp.int32)
    emb = _build(N, T, D)
    out = emb(ids_flat, tok_table, pos_table)
    return out.reshape(B, T, D)


# speedup vs baseline: 1.0616x; 1.0616x over previous
import functools

import jax
import jax.numpy as jnp
from jax import lax
from jax.experimental import pallas as pl
from jax.experimental.pallas import tpu as pltpu
from jax.experimental.pallas import tpu_sc as plsc

_NC = 2
_NS = 16
_NW = _NC * _NS
_LANES = 16
_NBUF = 2


def _build(N, T, D):
    rows_per_w = N // _NW
    seqs_per_w = rows_per_w // T
    SEQ_PER_CHUNK = 2
    CH_ROWS = SEQ_PER_CHUNK * T
    n_chunks = seqs_per_w // SEQ_PER_CHUNK
    G = 80
    n_g = CH_ROWS // G

    mesh = plsc.VectorSubcoreMesh(
        core_axis_name="c", subcore_axis_name="s",
        num_cores=_NC, num_subcores=_NS,
    )

    @functools.partial(
        pl.kernel,
        out_type=jax.ShapeDtypeStruct((N, D), jnp.float32),
        mesh=mesh,
        scratch_types=[
            pltpu.VMEM((T, D), jnp.float32),
            pltpu.VMEM((rows_per_w,), jnp.int32),
            [pltpu.VMEM((CH_ROWS, D), jnp.float32) for _ in range(_NBUF)],
            pltpu.SemaphoreType.DMA,
            [pltpu.SemaphoreType.DMA for _ in range(_NBUF)],
        ],
        compiler_params=pltpu.CompilerParams(use_tc_tiling_on_sc=False),
    )
    def emb_kernel(ids_hbm, tok_hbm, pos_hbm, out_hbm,
                   pos_v, idx_v, rows, gsem, ssem):
        wid = lax.axis_index("s") * _NC + lax.axis_index("c")
        base = pl.multiple_of(wid * rows_per_w, CH_ROWS)
        pltpu.sync_copy(ids_hbm.at[pl.ds(base, rows_per_w)], idx_v)
        pltpu.sync_copy(pos_hbm, pos_v)

        def gather(c, b):
            copies = []
            for g in range(n_g):
                off = pl.multiple_of(c * CH_ROWS + g * G, G)
                cp = pltpu.make_async_copy(
                    tok_hbm.at[idx_v.at[pl.ds(off, G)]],
                    rows[b].at[pl.ds(g * G, G)],
                    gsem,
                )
                cp.start()
                copies.append(cp)
            for cp in copies:
                cp.wait()

        def add_pos(b):
            @plsc.parallel_loop(0, T, unroll=8)
            def _(t):
                for cc in range(D // _LANES):
                    v = pos_v[t, pl.ds(cc * _LANES, _LANES)]
                    for s in range(SEQ_PER_CHUNK):
                        plsc.addupdate(
                            rows[b].at[s * T + t, pl.ds(cc * _LANES, _LANES)], v
                        )

        def store_copy(c, b):
            row0 = pl.multiple_of(base + c * CH_ROWS, CH_ROWS)
            return pltpu.make_async_copy(
                rows[b], out_hbm.at[pl.ds(row0, CH_ROWS)], ssem[b]
            )

        def slot(c, b, wait_prev):
            if wait_prev:
                store_copy(c - _NBUF, b).wait()
            gather(c, b)
            add_pos(b)
            store_copy(c, b).start()

        for b in range(_NBUF):
            slot(b, b, wait_prev=False)

        def group(gi, carry):
            c0 = _NBUF + gi * _NBUF
            for b in range(_NBUF):
                slot(c0 + b, b, wait_prev=True)
            return carry

        lax.fori_loop(0, n_chunks // _NBUF - 1, group, 0)

        for b in range(_NBUF):
            store_copy(n_chunks - _NBUF + b, b).wait()

    return emb_kernel


def kernel(token_ids, tok_table, pos_table):
    B, T = token_ids.shape
    V, D = tok_table.shape
    N = B * T
    ids_flat = token_ids.reshape(N).astype(jnp.int32)
    emb = _build(N, T, D)
    out = emb(ids_flat, tok_table, pos_table)
    return out.reshape(B, T, D)
